# Initial kernel scaffold; baseline (speedup 1.0000x reference)
import jax
import jax.numpy as jnp
from jax.experimental import pallas as pl


def kernel(data, channelindex):
    raise NotImplementedError("write your pallas kernel here")


# speedup vs baseline: 287.4580x; 287.4580x over previous
import functools

import jax
import jax.numpy as jnp
from jax import lax
from jax.experimental import pallas as pl
from jax.experimental.pallas import tpu as pltpu
from jax.experimental.pallas import tpu_sc as plsc

NUM_BANDS = 1000
NUM_CHANNELS = 3276800

NC = 2
NS = 16
NW = NC * NS
L = 16

PER_W = NUM_CHANNELS // NW
CHUNK = 12800
N_CHUNKS = PER_W // CHUNK


def _gather_body(data_hbm, idx_hbm, out_hbm, table_v, idx_v, out_v):
    wid = lax.axis_index("s") * NC + lax.axis_index("c")
    base = wid * PER_W
    pltpu.sync_copy(data_hbm, table_v)

    def chunk_body(g, carry):
        off = base + g * CHUNK

        pltpu.sync_copy(idx_hbm.at[pl.ds(off, CHUNK)], idx_v)

        def vec_body(i, c):
            sl = pl.ds(i * L, L)
            out_v[sl] = plsc.load_gather(table_v, [idx_v[sl]])
            return c

        lax.fori_loop(0, CHUNK // L, vec_body, 0, unroll=8)

        pltpu.sync_copy(out_v, out_hbm.at[pl.ds(off, CHUNK)])
        return carry

    lax.fori_loop(0, N_CHUNKS, chunk_body, 0)


@functools.partial(
    pl.kernel,
    out_type=jax.ShapeDtypeStruct((NUM_CHANNELS,), jnp.float32),
    mesh=plsc.VectorSubcoreMesh(core_axis_name="c", subcore_axis_name="s"),
    scratch_types=[
        pltpu.VMEM((NUM_BANDS,), jnp.float32),
        pltpu.VMEM((CHUNK,), jnp.int32),
        pltpu.VMEM((CHUNK,), jnp.float32),
    ],
    compiler_params=pltpu.CompilerParams(needs_layout_passes=False),
)
def _gather_call(data_hbm, idx_hbm, out_hbm, table_v, idx_v, out_v):
    _gather_body(data_hbm, idx_hbm, out_hbm, table_v, idx_v, out_v)


def kernel(data, channelindex):
    return _gather_call(data, channelindex.astype(jnp.int32))


# speedup vs baseline: 330.4305x; 1.1495x over previous
import functools

import jax
import jax.numpy as jnp
from jax import lax
from jax.experimental import pallas as pl
from jax.experimental.pallas import tpu as pltpu
from jax.experimental.pallas import tpu_sc as plsc

NUM_BANDS = 1000
NUM_CHANNELS = 3276800

NC = 2
NS = 16
NW = NC * NS
L = 16

PER_W = NUM_CHANNELS // NW
CHUNK = 12800
N_CHUNKS = PER_W // CHUNK


def _gather_body(data_hbm, idx_hbm, out_hbm, table_v,
                 i0, i1, o0, o1, si0, si1, so0, so1):
    idxb, outb = [i0, i1], [o0, o1]
    sin, sout = [si0, si1], [so0, so1]
    wid = lax.axis_index("s") * NC + lax.axis_index("c")
    base = wid * PER_W
    pltpu.sync_copy(data_hbm, table_v)

    def start_in(g):
        b = g % 2
        return pltpu.async_copy(
            idx_hbm.at[pl.ds(base + g * CHUNK, CHUNK)], idxb[b], sin[b])

    hin = {0: start_in(0)}
    hout = {}
    for g in range(N_CHUNKS):
        b = g % 2
        hin[g].wait()
        if g + 1 < N_CHUNKS:
            hin[g + 1] = start_in(g + 1)
        if g - 2 >= 0:
            hout[g - 2].wait()

        def vec_body(i, c):
            sl = pl.ds(i * L, L)
            outb[b][sl] = plsc.load_gather(table_v, [idxb[b][sl]])
            return c

        lax.fori_loop(0, CHUNK // L, vec_body, 0, unroll=8)

        hout[g] = pltpu.async_copy(
            outb[b], out_hbm.at[pl.ds(base + g * CHUNK, CHUNK)], sout[b])

    hout[N_CHUNKS - 2].wait()
    hout[N_CHUNKS - 1].wait()


@functools.partial(
    pl.kernel,
    out_type=jax.ShapeDtypeStruct((NUM_CHANNELS,), jnp.float32),
    mesh=plsc.VectorSubcoreMesh(core_axis_name="c", subcore_axis_name="s"),
    scratch_types=[
        pltpu.VMEM((NUM_BANDS,), jnp.float32),
        pltpu.VMEM((CHUNK,), jnp.int32),
        pltpu.VMEM((CHUNK,), jnp.int32),
        pltpu.VMEM((CHUNK,), jnp.float32),
        pltpu.VMEM((CHUNK,), jnp.float32),
        pltpu.SemaphoreType.DMA,
        pltpu.SemaphoreType.DMA,
        pltpu.SemaphoreType.DMA,
        pltpu.SemaphoreType.DMA,
    ],
    compiler_params=pltpu.CompilerParams(needs_layout_passes=False),
)
def _gather_call(data_hbm, idx_hbm, out_hbm, table_v,
                 i0, i1, o0, o1, si0, si1, so0, so1):
    _gather_body(data_hbm, idx_hbm, out_hbm, table_v,
                 i0, i1, o0, o1, si0, si1, so0, so1)


def kernel(data, channelindex):
    return _gather_call(data, channelindex.astype(jnp.int32))


# speedup vs baseline: 760.8487x; 2.3026x over previous
import functools

import jax
import jax.numpy as jnp
from jax import lax
from jax.experimental import pallas as pl
from jax.experimental.pallas import tpu as pltpu
from jax.experimental.pallas import tpu_sc as plsc

NUM_BANDS = 1000
NUM_CHANNELS = 3276800

NC = 2
NS = 16
NW = NC * NS
L = 16

PER_W = NUM_CHANNELS // NW
CHUNK = 12800
N_CHUNKS = PER_W // CHUNK


def _gather_body(data_hbm, idx_hbm, out_hbm, table_v,
                 i0, i1, o0, o1, si0, si1, so0, so1):
    idxb, outb = [i0, i1], [o0, o1]
    sin, sout = [si0, si1], [so0, so1]
    wid = lax.axis_index("s") * NC + lax.axis_index("c")
    base = wid * PER_W
    pltpu.sync_copy(data_hbm, table_v)

    def start_in(g):
        b = g % 2
        return pltpu.async_copy(
            idx_hbm.at[pl.ds(base + g * CHUNK, CHUNK)], idxb[b], sin[b])

    hin = {0: start_in(0)}
    hout = {}
    for g in range(N_CHUNKS):
        b = g % 2
        hin[g].wait()
        if g + 1 < N_CHUNKS:
            hin[g + 1] = start_in(g + 1)
        if g - 2 >= 0:
            hout[g - 2].wait()

        ib, ob = idxb[b], outb[b]

        @plsc.parallel_loop(0, CHUNK, step=L, unroll=8)
        def _(i):
            ob[pl.ds(i, L)] = plsc.load_gather(table_v, [ib[pl.ds(i, L)]])

        hout[g] = pltpu.async_copy(
            outb[b], out_hbm.at[pl.ds(base + g * CHUNK, CHUNK)], sout[b])

    hout[N_CHUNKS - 2].wait()
    hout[N_CHUNKS - 1].wait()


@functools.partial(
    pl.kernel,
    out_type=jax.ShapeDtypeStruct((NUM_CHANNELS,), jnp.float32),
    mesh=plsc.VectorSubcoreMesh(core_axis_name="c", subcore_axis_name="s"),
    scratch_types=[
        pltpu.VMEM((NUM_BANDS,), jnp.float32),
        pltpu.VMEM((CHUNK,), jnp.int32),
        pltpu.VMEM((CHUNK,), jnp.int32),
        pltpu.VMEM((CHUNK,), jnp.float32),
        pltpu.VMEM((CHUNK,), jnp.float32),
        pltpu.SemaphoreType.DMA,
        pltpu.SemaphoreType.DMA,
        pltpu.SemaphoreType.DMA,
        pltpu.SemaphoreType.DMA,
    ],
    compiler_params=pltpu.CompilerParams(needs_layout_passes=False),
)
def _gather_call(data_hbm, idx_hbm, out_hbm, table_v,
                 i0, i1, o0, o1, si0, si1, so0, so1):
    _gather_body(data_hbm, idx_hbm, out_hbm, table_v,
                 i0, i1, o0, o1, si0, si1, so0, so1)


def kernel(data, channelindex):
    return _gather_call(data, channelindex.astype(jnp.int32))
